# baseline (speedup 1.0000x reference)
import jax
import jax.numpy as jnp
from jax import lax
from jax.experimental import pallas as pl
from jax.experimental.pallas import tpu as pltpu
from jax.experimental.pallas import tpu_sc as plsc

NEG_SLOPE = 0.2
LN_EPS = 1e-5

NC = 2
NS = 16
LANES = 16
NW = NC * NS
K = 48
ACC_W = 144
NEG_BIG = -1e30


def _dense_body(x_ref, lnw_ref, lnb_ref, wl_ref, bl_ref, wr_ref, br_ref,
                aabs_ref, xl_ref, xr_ref, mb_ref):
    xb = x_ref[...]
    mu = jnp.mean(xb, axis=1, keepdims=True)
    d = xb - mu
    var = jnp.mean(d * d, axis=1, keepdims=True)
    xn = d * lax.rsqrt(var + LN_EPS) * lnw_ref[...] + lnb_ref[...]
    xn = jnp.maximum(xn, 0.0)
    xl = jnp.dot(xn, wl_ref[...], preferred_element_type=jnp.float32) \
        + bl_ref[...]
    xr = jnp.dot(xn, wr_ref[...], preferred_element_type=jnp.float32) \
        + br_ref[...]
    xl_ref[...] = xl
    xr_ref[...] = xr
    aabs = aabs_ref[...]
    umax = jnp.max(jnp.sum(jnp.abs(xl) * aabs, axis=1))
    wmax = jnp.max(jnp.sum(jnp.abs(xr) * aabs, axis=1))
    mb_ref[...] = jnp.concatenate(
        [jnp.full((1, 1, 16), umax, jnp.float32),
         jnp.full((1, 1, 16), wmax, jnp.float32)], axis=1)


def _combine_body(acc_ref, bias_ref, out_ref):
    a = acc_ref[0] + acc_ref[1]
    num = a[:, :128]
    den = jnp.maximum(a[:, 128:129], 1e-38)
    out_ref[...] = num / den + bias_ref[...]


def _make_pass(n, n_total_edges, nchunk, nblocks):
    nc8 = 128 // LANES
    ki = K // LANES
    rows_per_tile = n // NS
    zrows = 125

    def body(xl_hbm, xr_hbm, edges_hbm, att_hbm, mb_hbm,
             acc_hbm,
             att_v, mb_v, rr, xl_rows, xr_rows, rows, pbuf, evsplat, acc_sh,
             si0, si1, si2, si3, sgx0, sgx1, sgy0, sgy1, ss0, ss1):
        si = (si0, si1, si2, si3)
        sgx = (sgx0, sgx1)
        sgy = (sgy0, sgy1)
        ss = (ss0, ss1)
        cid = lax.axis_index("c")
        sid = lax.axis_index("s")
        wid = sid * NC + cid
        chunk0 = wid * nchunk

        pltpu.sync_copy(att_hbm, att_v)
        pltpu.sync_copy(mb_hbm, mb_v)
        uv = mb_v[0, 0]
        wv = mb_v[0, 1]
        for i in range(1, nblocks):
            uv = jnp.maximum(uv, mb_v[i, 0])
            wv = jnp.maximum(wv, mb_v[i, 1])
        gmax = jnp.max(uv) + jnp.max(wv)
        att_regs = [att_v[pl.ds(c * LANES, LANES)] for c in range(nc8)]
        lane = lax.iota(jnp.int32, LANES)

        def zero_row(r, _):
            for bb in range(2):
                for c in range(ACC_W // LANES):
                    rows[bb, r, pl.ds(c * LANES, LANES)] = jnp.zeros(
                        (LANES,), jnp.float32)
            return 0
        lax.fori_loop(0, K, zero_row, 0)
        nz = zrows // K + (1 if zrows % K else 0)
        for j in range(rows_per_tile // zrows):
            for h in range(nz):
                r0 = sid * rows_per_tile + j * zrows + h * K
                cnt = min(K, zrows - h * K)
                pltpu.sync_copy(rows.at[0, pl.ds(0, cnt)],
                                acc_sh.at[pl.ds(r0, cnt)])
        plsc.subcore_barrier()

        def fire_idx(kk, q):
            pltpu.async_copy(edges_hbm.at[chunk0 + kk], rr.at[q], si[q])

        def wait_idx(q):
            pltpu.make_async_copy(edges_hbm.at[chunk0], rr.at[q],
                                  si[q]).wait()

        def fire_gather(b, q):
            pltpu.async_copy(xl_hbm.at[rr.at[q, 0]], xl_rows.at[b], sgx[b])
            pltpu.async_copy(xr_hbm.at[rr.at[q, 1]], xr_rows.at[b], sgy[b])

        for q in range(4):
            fire_idx(q, q)
        wait_idx(0)
        fire_gather(0, 0)

        def quad(p, _):
            for j in range(4):
                k = p * 4 + j
                q = j
                b = j % 2
                pltpu.make_async_copy(xl_hbm.at[rr.at[q, 0]],
                                      xl_rows.at[b], sgx[b]).wait()
                pltpu.make_async_copy(xr_hbm.at[rr.at[q, 1]],
                                      xr_rows.at[b], sgy[b]).wait()

                @pl.when(k >= 2)
                def _():
                    pltpu.make_async_copy(acc_hbm.at[cid, pl.ds(0, K)],
                                          rows.at[b], ss[b]).wait()

                @pl.when(k + 2 < nchunk)
                def _():
                    fire_idx(k + 2, (j + 2) % 4)

                @pl.when(k + 1 < nchunk)
                def _():
                    qn = (j + 1) % 4
                    wait_idx(qn)
                    fire_gather(1 - b, qn)

                ebase = (chunk0 + k) * K

                @plsc.parallel_loop(0, K, step=1, unroll=8)
                def _(e):
                    acc = None
                    for c in range(nc8):
                        sl = pl.ds(c * LANES, LANES)
                        z = xl_rows[b, e, sl] + xr_rows[b, e, sl]
                        zl = jnp.where(z >= 0.0, z, NEG_SLOPE * z)
                        t = att_regs[c] * zl
                        acc = t if acc is None else acc + t
                    pbuf[pl.ds(e * LANES, LANES)] = acc

                def grp2(g, _):
                    s = None
                    for c in range(LANES):
                        idx = g * (LANES * LANES) + lane * LANES + c
                        t = plsc.load_gather(pbuf, [idx])
                        s = t if s is None else s + t
                    eid = ebase + g * LANES + lane
                    s = jnp.where(eid < n_total_edges, s, NEG_BIG)
                    evv = jnp.exp(s - gmax)
                    for l in range(LANES):
                        evsplat[g * LANES + l] = jnp.full(
                            (LANES,), evv[l], jnp.float32)
                    return 0

                lax.fori_loop(0, ki, grp2, 0)

                @plsc.parallel_loop(0, K, step=1, unroll=8)
                def _(e):
                    sv = evsplat[e]
                    for c in range(nc8):
                        sl = pl.ds(c * LANES, LANES)
                        rows[b, e, sl] = sv * xl_rows[b, e, sl]
                    rows[b, e, pl.ds(128, LANES)] = jnp.where(
                        lane == 0, sv, 0.0)
                pltpu.async_copy(rows.at[b], acc_sh.at[rr.at[q, 1]],
                                 ss[b], add=True)
            return 0

        lax.fori_loop(0, nchunk // 4, quad, 0)
        for b in range(2):
            pltpu.make_async_copy(acc_hbm.at[cid, pl.ds(0, K)],
                                  rows.at[b], ss[b]).wait()
        plsc.subcore_barrier()
        for j in range(rows_per_tile // zrows):
            for h in range(nz):
                r0 = sid * rows_per_tile + j * zrows + h * K
                cnt = min(K, zrows - h * K)
                pltpu.sync_copy(acc_sh.at[pl.ds(r0, cnt)],
                                acc_hbm.at[cid, pl.ds(r0, cnt)])

    return body


def kernel(x, edge_index, ln_w, ln_b, W_l, b_l, W_r, b_r, att, bias):
    n, d = x.shape
    e = edge_index.shape[1]
    c_out = W_l.shape[0]
    assert d == 128 and c_out == 128 and att.shape == (1, 128)
    assert n % (NS * 125) == 0

    etot = e + n
    nchunk = (etot + NW * K - 1) // (NW * K)
    nchunk = ((nchunk + 3) // 4) * 4
    per_w = nchunk * K
    ep = per_w * NW
    epc = ep // K
    pad = ep - etot

    rb = 1000
    grid = (n // rb,)
    nblocks = n // rb
    f32 = jnp.float32
    att_flat = att.reshape(-1)
    xl, xr, mb = pl.pallas_call(
        _dense_body,
        grid=grid,
        in_specs=[
            pl.BlockSpec((rb, d), lambda i: (i, 0)),
            pl.BlockSpec((1, d), lambda i: (0, 0)),
            pl.BlockSpec((1, d), lambda i: (0, 0)),
            pl.BlockSpec((d, c_out), lambda i: (0, 0)),
            pl.BlockSpec((1, c_out), lambda i: (0, 0)),
            pl.BlockSpec((d, c_out), lambda i: (0, 0)),
            pl.BlockSpec((1, c_out), lambda i: (0, 0)),
            pl.BlockSpec((1, c_out), lambda i: (0, 0)),
        ],
        out_specs=[
            pl.BlockSpec((rb, c_out), lambda i: (i, 0)),
            pl.BlockSpec((rb, c_out), lambda i: (i, 0)),
            pl.BlockSpec((1, 2, 16), lambda i: (i, 0, 0)),
        ],
        out_shape=[
            jax.ShapeDtypeStruct((n, c_out), f32),
            jax.ShapeDtypeStruct((n, c_out), f32),
            jax.ShapeDtypeStruct((nblocks, 2, 16), f32),
        ],
    )(x, ln_w.reshape(1, d), ln_b.reshape(1, d),
      W_l.T, b_l.reshape(1, c_out), W_r.T, b_r.reshape(1, c_out),
      jnp.abs(att_flat).reshape(1, c_out))

    loop = jnp.arange(n, dtype=jnp.int32)
    zpad = jnp.zeros((pad,), jnp.int32)
    src = jnp.concatenate([edge_index[0], loop, zpad]).reshape(epc, K)
    dst = jnp.concatenate([edge_index[1], loop, zpad]).reshape(epc, K)
    edges2 = jnp.stack([src, dst], axis=1)

    mesh = plsc.VectorSubcoreMesh(core_axis_name="c", subcore_axis_name="s",
                                  num_cores=NC, num_subcores=NS)
    sc_params = pltpu.CompilerParams(needs_layout_passes=False,
                                     use_tc_tiling_on_sc=False)

    sc_pass = pl.kernel(
        _make_pass(n, etot, nchunk, nblocks),
        out_type=jax.ShapeDtypeStruct((NC, n, ACC_W), f32),
        mesh=mesh,
        scratch_types=(
            pltpu.VMEM((128,), f32),
            pltpu.VMEM((nblocks, 2, 16), f32),
            pltpu.VMEM((4, 2, K), jnp.int32),
            pltpu.VMEM((2, K, 128), f32),
            pltpu.VMEM((2, K, 128), f32),
            pltpu.VMEM((2, K, ACC_W), f32),
            pltpu.VMEM((K * LANES,), f32),
            pltpu.VMEM((K, LANES), f32),
            pltpu.VMEM_SHARED((n, ACC_W), f32),
            pltpu.SemaphoreType.DMA,
            pltpu.SemaphoreType.DMA,
            pltpu.SemaphoreType.DMA,
            pltpu.SemaphoreType.DMA,
            pltpu.SemaphoreType.DMA,
            pltpu.SemaphoreType.DMA,
            pltpu.SemaphoreType.DMA,
            pltpu.SemaphoreType.DMA,
            pltpu.SemaphoreType.DMA,
            pltpu.SemaphoreType.DMA,
        ),
        compiler_params=sc_params,
    )
    acc2 = sc_pass(xl, xr, edges2, att_flat, mb)

    out = pl.pallas_call(
        _combine_body,
        grid=grid,
        in_specs=[
            pl.BlockSpec((NC, rb, ACC_W), lambda i: (0, i, 0)),
            pl.BlockSpec((1, c_out), lambda i: (0, 0)),
        ],
        out_specs=pl.BlockSpec((rb, c_out), lambda i: (i, 0)),
        out_shape=jax.ShapeDtypeStruct((n, c_out), f32),
    )(acc2, bias.reshape(1, c_out))
    return out
